# baseline (speedup 1.0000x reference)
import jax
import jax.numpy as jnp
from jax.experimental import pallas as pl
from jax.experimental.pallas import tpu as pltpu

N = 4096
D = 64
BR = 256
NB = N // BR
DEPTH = 4
EPS = 1e-5


def _gnn_kernel(adj_ref, x_ref, wt_ref, b_ref, gm_ref, bt_ref,
                out_ref, at_ref, ht_ref, d_ref, c_ref, buf_ref, sem_ref):
    i = pl.program_id(0)

    def copy(j):
        return pltpu.make_async_copy(
            adj_ref.at[pl.ds(j * BR, BR), :],
            buf_ref.at[j % DEPTH],
            sem_ref.at[j % DEPTH])

    @pl.when(i == 0)
    def _prologue():
        for j in range(DEPTH):
            copy(j).start()

    @pl.when(i < NB)
    def _stream():
        copy(i).wait()
        blk = buf_ref[i % DEPTH]
        deg_raw = jnp.sum(blk, axis=1, keepdims=True)
        at_ref[i] = jnp.transpose(blk.astype(jnp.bfloat16))
        base = i * BR
        sub = at_ref[i, pl.ds(base, BR), :]
        ri = jax.lax.broadcasted_iota(jnp.int32, (BR, BR), 0)
        ci = jax.lax.broadcasted_iota(jnp.int32, (BR, BR), 1)
        diag = jnp.sum(jnp.where(ri == ci, sub.astype(jnp.float32), 0.0),
                       axis=0, keepdims=True)
        c = 1.0 - diag
        deg = jnp.transpose(deg_raw) + c
        d_ref[:, pl.ds(base, BR)] = jnp.maximum(deg, 1.0) ** -0.5
        c_ref[:, pl.ds(base, BR)] = c

        @pl.when(i + DEPTH < NB)
        def _issue_next():
            copy(i + DEPTH).start()

    @pl.when(i == 0)
    def _init_h():
        ht_ref[...] = jnp.transpose(x_ref[...])

    def _layer(l, write_out):
        ht = ht_ref[...]
        d = d_ref[...]
        cv = c_ref[...]
        hwt = jnp.dot(wt_ref[l].astype(jnp.bfloat16), ht.astype(jnp.bfloat16),
                      preferred_element_type=jnp.float32)
        gt = hwt * d
        g_hi = gt.astype(jnp.bfloat16)
        g_lo = (gt - g_hi.astype(jnp.float32)).astype(jnp.bfloat16)
        ghl = jnp.concatenate([g_hi, g_lo], axis=0)
        bias = b_ref[l]

        for cb in range(NB):
            sl = slice(cb * BR, (cb + 1) * BR)
            ag2 = jnp.dot(ghl, at_ref[cb],
                          preferred_element_type=jnp.float32)
            ag = ag2[:D, :] + ag2[D:, :]
            ag = ag + cv[:, sl] * gt[:, sl]
            o = ag * d[:, sl] + bias
            ht_ref[:, sl] = jnp.maximum(o, 0.0)

        o_full = ht_ref[...]
        mean = jnp.mean(o_full, axis=1, keepdims=True)
        var = jnp.mean((o_full - mean) ** 2, axis=1, keepdims=True)
        scale = gm_ref[l] * jax.lax.rsqrt(var + EPS)
        shift = bt_ref[l] - mean * scale
        hn = o_full * scale + shift
        ht_ref[...] = hn
        if write_out:
            out_ref[...] = jnp.transpose(hn)

    @pl.when(i == NB)
    def _l1():
        _layer(0, False)

    @pl.when(i == NB + 1)
    def _l2():
        _layer(1, False)

    @pl.when(i == NB + 2)
    def _l3():
        _layer(2, True)


def kernel(x, adj, W1, b1, g1, be1, W2, b2, g2, be2, W3, b3, g3, be3):
    WT = jnp.stack([W1.T, W2.T, W3.T])
    b = jnp.stack([b1, b2, b3])[:, :, None]
    gm = jnp.stack([g1, g2, g3])[:, :, None]
    bt = jnp.stack([be1, be2, be3])[:, :, None]

    return pl.pallas_call(
        _gnn_kernel,
        grid=(NB + 3,),
        in_specs=[
            pl.BlockSpec(memory_space=pl.ANY),
            pl.BlockSpec((N, D), lambda i: (0, 0)),
            pl.BlockSpec((3, D, D), lambda i: (0, 0, 0)),
            pl.BlockSpec((3, D, 1), lambda i: (0, 0, 0)),
            pl.BlockSpec((3, D, 1), lambda i: (0, 0, 0)),
            pl.BlockSpec((3, D, 1), lambda i: (0, 0, 0)),
        ],
        out_specs=pl.BlockSpec((N, D), lambda i: (0, 0)),
        out_shape=jax.ShapeDtypeStruct((N, D), jnp.float32),
        scratch_shapes=[
            pltpu.VMEM((NB, N, BR), jnp.bfloat16),
            pltpu.VMEM((D, N), jnp.float32),
            pltpu.VMEM((1, N), jnp.float32),
            pltpu.VMEM((1, N), jnp.float32),
            pltpu.VMEM((DEPTH, BR, N), jnp.float32),
            pltpu.SemaphoreType.DMA((DEPTH,)),
        ],
        compiler_params=pltpu.CompilerParams(
            dimension_semantics=("arbitrary",),
            vmem_limit_bytes=60 * 1024 * 1024,
        ),
    )(adj, x, WT, b, gm, bt)
